# Initial kernel scaffold; baseline (speedup 1.0000x reference)
import jax
import jax.numpy as jnp
from jax.experimental import pallas as pl


def kernel(inputs, fc_pos_W, fc_pos_b, blocks_W0, blocks_b0, blocks_W1, blocks_b1, blocks_Ws, fc_c_W, fc_c_b):
    raise NotImplementedError("write your pallas kernel here")


# speedup vs baseline: 7.1455x; 7.1455x over previous
import functools

import jax
import jax.numpy as jnp
from jax import lax
from jax.experimental import pallas as pl
from jax.experimental.pallas import tpu as pltpu
from jax.experimental.pallas import tpu_sc as plsc

RESO = 128
R2 = RESO * RESO
HID = 32
CDIM = 32
NBLK = 5

CHT = 2048
SC_CH = 2048
CPT = 4


def _s0_body(x_ref, W_ref, b_ref, W0_ref, b0_ref, W1_ref, b1_ref, Ws_ref,
             net_ref, idx_ref):
    x = x_ref[0]
    xi = (x[:, 0] * float(RESO)).astype(jnp.int32)
    yi = (x[:, 1] * float(RESO)).astype(jnp.int32)
    idx_ref[0, 0] = xi + RESO * yi

    W = W_ref[...]
    net = lax.dot_general(W, x, (((1,), (1,)), ((), ())),
                          preferred_element_type=jnp.float32)
    net = net + b_ref[...]
    xr = jnp.maximum(net, 0.0)
    h = jnp.dot(W0_ref[...], xr, preferred_element_type=jnp.float32) + b0_ref[...]
    d = jnp.dot(W1_ref[...], jnp.maximum(h, 0.0),
                preferred_element_type=jnp.float32) + b1_ref[...]
    net_ref[0] = jnp.dot(Ws_ref[...], net, preferred_element_type=jnp.float32) + d


def _blk_body(net_ref, pool_ref, W0_ref, b0_ref, W1_ref, b1_ref, Ws_ref, out_ref):
    x = jnp.concatenate([net_ref[0], pool_ref[0]], axis=0)
    xr = jnp.maximum(x, 0.0)
    h = jnp.dot(W0_ref[...], xr, preferred_element_type=jnp.float32) + b0_ref[...]
    d = jnp.dot(W1_ref[...], jnp.maximum(h, 0.0),
                preferred_element_type=jnp.float32) + b1_ref[...]
    out_ref[0] = jnp.dot(Ws_ref[...], x, preferred_element_type=jnp.float32) + d


def _s5_body(net_ref, W_ref, b_ref, out_ref):
    xr = jnp.maximum(net_ref[0], 0.0)
    out_ref[0] = jnp.dot(W_ref[...], xr, preferred_element_type=jnp.float32) + b_ref[...]


def _full(shape):
    return pl.BlockSpec(shape, lambda b, i: (0,) * len(shape))


def _tc_s0(inputs, fc_pos_W, fc_pos_b, W0, b0, W1, b1, Ws):
    B, N, _ = inputs.shape
    grid = (B, N // CHT)
    return pl.pallas_call(
        _s0_body,
        grid=grid,
        in_specs=[
            pl.BlockSpec((1, CHT, 3), lambda b, i: (b, i, 0)),
            _full(fc_pos_W.shape), _full(fc_pos_b.shape),
            _full(W0.shape), _full(b0.shape),
            _full(W1.shape), _full(b1.shape),
            _full(Ws.shape),
        ],
        out_specs=[
            pl.BlockSpec((1, HID, CHT), lambda b, i: (b, 0, i)),
            pl.BlockSpec((1, 1, CHT), lambda b, i: (b, 0, i)),
        ],
        out_shape=[
            jax.ShapeDtypeStruct((B, HID, N), jnp.float32),
            jax.ShapeDtypeStruct((B, 1, N), jnp.int32),
        ],
    )(inputs, fc_pos_W, fc_pos_b, W0, b0, W1, b1, Ws)


def _tc_block(netT, poolT, W0, b0, W1, b1, Ws):
    B, _, N = netT.shape
    grid = (B, N // CHT)
    return pl.pallas_call(
        _blk_body,
        grid=grid,
        in_specs=[
            pl.BlockSpec((1, HID, CHT), lambda b, i: (b, 0, i)),
            pl.BlockSpec((1, HID, CHT), lambda b, i: (b, 0, i)),
            _full(W0.shape), _full(b0.shape),
            _full(W1.shape), _full(b1.shape),
            _full(Ws.shape),
        ],
        out_specs=pl.BlockSpec((1, HID, CHT), lambda b, i: (b, 0, i)),
        out_shape=jax.ShapeDtypeStruct((B, HID, N), jnp.float32),
    )(netT, poolT, W0, b0, W1, b1, Ws)


def _tc_s5(netT, W, b):
    B, _, N = netT.shape
    grid = (B, N // CHT)
    return pl.pallas_call(
        _s5_body,
        grid=grid,
        in_specs=[
            pl.BlockSpec((1, HID, CHT), lambda b, i: (b, 0, i)),
            _full(W.shape), _full(b.shape),
        ],
        out_specs=pl.BlockSpec((1, CDIM, CHT), lambda b, i: (b, 0, i)),
        out_shape=jax.ShapeDtypeStruct((B, CDIM, N), jnp.float32),
    )(netT, W, b)


def _sc_mesh():
    return plsc.VectorSubcoreMesh(core_axis_name="c", subcore_axis_name="s")


def _unique_passes(ci, fn):
    cnt, _ = plsc.scan_count(ci)
    kmin = jnp.min(cnt)
    kmax = jnp.max(cnt)

    def pass_body(k, _):
        fn(cnt == k)
        return 0

    lax.fori_loop(kmin, kmax + 1, pass_body, 0)


def _pool_kernel(B, N):
    n_units = B * (HID // CPT)
    units_per_tile = n_units // 32
    n_chunks = N // SC_CH
    sub = SC_CH // 16

    @functools.partial(
        pl.kernel,
        out_type=jax.ShapeDtypeStruct((B, HID, N), jnp.float32),
        mesh=_sc_mesh(),
        compiler_params=pltpu.CompilerParams(needs_layout_passes=False),
        scratch_types=[
            pltpu.VMEM((CPT * R2,), jnp.float32),
            pltpu.VMEM((SC_CH,), jnp.int32),
            pltpu.VMEM((CPT, SC_CH), jnp.float32),
            pltpu.VMEM((CPT, SC_CH), jnp.float32),
        ],
    )
    def pool(net_hbm, idx_hbm, out_hbm, acc, idxb, valb, outb):
        wid = lax.axis_index("s") * 2 + lax.axis_index("c")

        def unit_body(u, _):
            b = u // (HID // CPT)
            cg = (u % (HID // CPT)) * CPT

            def init(i, _):
                acc[pl.ds(i * 16, 16)] = jnp.full((16,), -jnp.inf, jnp.float32)
                return 0
            lax.fori_loop(0, CPT * R2 // 16, init, 0)

            def mchunk(mc, _):
                p0 = mc * SC_CH
                pltpu.sync_copy(idx_hbm.at[b, 0, pl.ds(p0, SC_CH)], idxb)
                pltpu.sync_copy(net_hbm.at[b, pl.ds(cg, CPT), pl.ds(p0, SC_CH)], valb)

                def chunk(j, _):
                    ci = idxb[pl.ds(j * 16, 16)]

                    def upd(sel):
                        for c in range(CPT):
                            v = valb[c, pl.ds(j * 16, 16)]
                            cic = ci + (c * R2)
                            cur = plsc.load_gather(acc, [cic], mask=sel)
                            plsc.store_scatter(acc, [cic],
                                               jnp.maximum(cur, v), mask=sel)

                    _unique_passes(ci, upd)
                    return 0
                lax.fori_loop(0, sub, chunk, 0)
                return 0
            lax.fori_loop(0, n_chunks, mchunk, 0)

            def gchunk(mc, _):
                p0 = mc * SC_CH
                pltpu.sync_copy(idx_hbm.at[b, 0, pl.ds(p0, SC_CH)], idxb)

                def chunk(j, _):
                    ci = idxb[pl.ds(j * 16, 16)]
                    for c in range(CPT):
                        outb[c, pl.ds(j * 16, 16)] = plsc.load_gather(
                            acc, [ci + (c * R2)])
                    return 0
                lax.fori_loop(0, sub, chunk, 0)
                pltpu.sync_copy(outb, out_hbm.at[b, pl.ds(cg, CPT), pl.ds(p0, SC_CH)])
                return 0
            lax.fori_loop(0, n_chunks, gchunk, 0)
            return 0

        lax.fori_loop(wid * units_per_tile, (wid + 1) * units_per_tile, unit_body, 0)

    return pool


def _mean_kernel(B, N):
    n_units = B * (CDIM // CPT)
    units_per_tile = n_units // 32
    n_chunks = N // SC_CH
    sub = SC_CH // 16

    @functools.partial(
        pl.kernel,
        out_type=jax.ShapeDtypeStruct((B, CDIM * R2), jnp.float32),
        mesh=_sc_mesh(),
        compiler_params=pltpu.CompilerParams(needs_layout_passes=False),
        scratch_types=[
            pltpu.VMEM(((CPT + 1) * R2,), jnp.float32),
            pltpu.VMEM((SC_CH,), jnp.int32),
            pltpu.VMEM((CPT, SC_CH), jnp.float32),
        ],
    )
    def mean(c_hbm, idx_hbm, out_hbm, acc, idxb, valb):
        wid = lax.axis_index("s") * 2 + lax.axis_index("c")

        def unit_body(u, _):
            b = u // (CDIM // CPT)
            cg = (u % (CDIM // CPT)) * CPT

            def init(i, _):
                acc[pl.ds(i * 16, 16)] = jnp.zeros((16,), jnp.float32)
                return 0
            lax.fori_loop(0, (CPT + 1) * R2 // 16, init, 0)

            def mchunk(mc, _):
                p0 = mc * SC_CH
                pltpu.sync_copy(idx_hbm.at[b, 0, pl.ds(p0, SC_CH)], idxb)
                pltpu.sync_copy(c_hbm.at[b, pl.ds(cg, CPT), pl.ds(p0, SC_CH)], valb)

                def chunk(j, _):
                    ci = idxb[pl.ds(j * 16, 16)]
                    ones = jnp.ones((16,), jnp.float32)

                    def upd(sel):
                        for c in range(CPT):
                            v = valb[c, pl.ds(j * 16, 16)]
                            cic = ci + (c * R2)
                            cur = plsc.load_gather(acc, [cic], mask=sel)
                            plsc.store_scatter(acc, [cic], cur + v, mask=sel)
                        cntc = ci + (CPT * R2)
                        curc = plsc.load_gather(acc, [cntc], mask=sel)
                        plsc.store_scatter(acc, [cntc], curc + ones, mask=sel)

                    _unique_passes(ci, upd)
                    return 0
                lax.fori_loop(0, sub, chunk, 0)
                return 0
            lax.fori_loop(0, n_chunks, mchunk, 0)

            def fin(i, _):
                n = acc[pl.ds(CPT * R2 + i * 16, 16)]
                inv = 1.0 / jnp.maximum(n, 1.0)
                for c in range(CPT):
                    s = acc[pl.ds(c * R2 + i * 16, 16)]
                    acc[pl.ds(c * R2 + i * 16, 16)] = s * inv
                return 0
            lax.fori_loop(0, R2 // 16, fin, 0)

            pltpu.sync_copy(acc.at[pl.ds(0, CPT * R2)],
                            out_hbm.at[b, pl.ds(cg * R2, CPT * R2)])
            return 0

        lax.fori_loop(wid * units_per_tile, (wid + 1) * units_per_tile, unit_body, 0)

    return mean


def kernel(inputs, fc_pos_W, fc_pos_b, blocks_W0, blocks_b0, blocks_W1,
           blocks_b1, blocks_Ws, fc_c_W, fc_c_b):
    B, N, _ = inputs.shape

    b_pos = fc_pos_b.reshape(-1, 1)
    b0 = blocks_b0.reshape(NBLK, -1, 1)
    b1 = blocks_b1.reshape(NBLK, -1, 1)
    b_c = fc_c_b.reshape(-1, 1)

    netT, idx = _tc_s0(inputs, fc_pos_W, b_pos,
                       blocks_W0[0], b0[0], blocks_W1[0], b1[0], blocks_Ws[0])

    pool = _pool_kernel(B, N)
    for i in range(1, NBLK):
        poolT = pool(netT, idx)
        netT = _tc_block(netT, poolT, blocks_W0[i], b0[i], blocks_W1[i],
                         b1[i], blocks_Ws[i])

    cT = _tc_s5(netT, fc_c_W, b_c)
    plane = _mean_kernel(B, N)(cT, idx)
    return plane.reshape(B, CDIM, RESO, RESO)


# speedup vs baseline: 7.9317x; 1.1100x over previous
import functools

import jax
import jax.numpy as jnp
from jax import lax
from jax.experimental import pallas as pl
from jax.experimental.pallas import tpu as pltpu
from jax.experimental.pallas import tpu_sc as plsc

RESO = 128
R2 = RESO * RESO
HID = 32
CDIM = 32
NBLK = 5

CHT = 2048
SC_CH = 2048
CPT = 4


def _s0_body(x_ref, W_ref, b_ref, W0_ref, b0_ref, W1_ref, b1_ref, Ws_ref,
             net_ref, idx_ref):
    x = x_ref[0]
    xi = (x[:, 0] * float(RESO)).astype(jnp.int32)
    yi = (x[:, 1] * float(RESO)).astype(jnp.int32)
    idx_ref[0, 0] = xi + RESO * yi

    W = W_ref[...]
    net = lax.dot_general(W, x, (((1,), (1,)), ((), ())),
                          preferred_element_type=jnp.float32)
    net = net + b_ref[...]
    xr = jnp.maximum(net, 0.0)
    h = jnp.dot(W0_ref[...], xr, preferred_element_type=jnp.float32) + b0_ref[...]
    d = jnp.dot(W1_ref[...], jnp.maximum(h, 0.0),
                preferred_element_type=jnp.float32) + b1_ref[...]
    net_ref[0] = jnp.dot(Ws_ref[...], net, preferred_element_type=jnp.float32) + d


def _blk_body(net_ref, pool_ref, W0_ref, b0_ref, W1_ref, b1_ref, Ws_ref, out_ref):
    x = jnp.concatenate([net_ref[0], pool_ref[0]], axis=0)
    xr = jnp.maximum(x, 0.0)
    h = jnp.dot(W0_ref[...], xr, preferred_element_type=jnp.float32) + b0_ref[...]
    d = jnp.dot(W1_ref[...], jnp.maximum(h, 0.0),
                preferred_element_type=jnp.float32) + b1_ref[...]
    out_ref[0] = jnp.dot(Ws_ref[...], x, preferred_element_type=jnp.float32) + d


def _s5_body(net_ref, W_ref, b_ref, out_ref):
    xr = jnp.maximum(net_ref[0], 0.0)
    out_ref[0] = jnp.dot(W_ref[...], xr, preferred_element_type=jnp.float32) + b_ref[...]


def _full(shape):
    return pl.BlockSpec(shape, lambda b, i: (0,) * len(shape))


def _tc_s0(inputs, fc_pos_W, fc_pos_b, W0, b0, W1, b1, Ws):
    B, N, _ = inputs.shape
    grid = (B, N // CHT)
    return pl.pallas_call(
        _s0_body,
        grid=grid,
        in_specs=[
            pl.BlockSpec((1, CHT, 3), lambda b, i: (b, i, 0)),
            _full(fc_pos_W.shape), _full(fc_pos_b.shape),
            _full(W0.shape), _full(b0.shape),
            _full(W1.shape), _full(b1.shape),
            _full(Ws.shape),
        ],
        out_specs=[
            pl.BlockSpec((1, HID, CHT), lambda b, i: (b, 0, i)),
            pl.BlockSpec((1, 1, CHT), lambda b, i: (b, 0, i)),
        ],
        out_shape=[
            jax.ShapeDtypeStruct((B, HID, N), jnp.float32),
            jax.ShapeDtypeStruct((B, 1, N), jnp.int32),
        ],
    )(inputs, fc_pos_W, fc_pos_b, W0, b0, W1, b1, Ws)


def _tc_block(netT, poolT, W0, b0, W1, b1, Ws):
    B, _, N = netT.shape
    grid = (B, N // CHT)
    return pl.pallas_call(
        _blk_body,
        grid=grid,
        in_specs=[
            pl.BlockSpec((1, HID, CHT), lambda b, i: (b, 0, i)),
            pl.BlockSpec((1, HID, CHT), lambda b, i: (b, 0, i)),
            _full(W0.shape), _full(b0.shape),
            _full(W1.shape), _full(b1.shape),
            _full(Ws.shape),
        ],
        out_specs=pl.BlockSpec((1, HID, CHT), lambda b, i: (b, 0, i)),
        out_shape=jax.ShapeDtypeStruct((B, HID, N), jnp.float32),
    )(netT, poolT, W0, b0, W1, b1, Ws)


def _tc_s5(netT, W, b):
    B, _, N = netT.shape
    grid = (B, N // CHT)
    return pl.pallas_call(
        _s5_body,
        grid=grid,
        in_specs=[
            pl.BlockSpec((1, HID, CHT), lambda b, i: (b, 0, i)),
            _full(W.shape), _full(b.shape),
        ],
        out_specs=pl.BlockSpec((1, CDIM, CHT), lambda b, i: (b, 0, i)),
        out_shape=jax.ShapeDtypeStruct((B, CDIM, N), jnp.float32),
    )(netT, W, b)


def _sc_mesh():
    return plsc.VectorSubcoreMesh(core_axis_name="c", subcore_axis_name="s")


def _sc_cp():
    return pltpu.CompilerParams(needs_layout_passes=False)


def _max_passes(acc, ci, cnt, km, valb, j):
    def pass_body(k, _):
        sel = cnt == k
        for c in range(CPT):
            v = valb[c, pl.ds(j * 16, 16)]
            cic = ci + (c * R2)
            cur = plsc.load_gather(acc, [cic], mask=sel)
            plsc.store_scatter(acc, [cic], jnp.maximum(cur, v), mask=sel)
        return 0
    lax.fori_loop(1, km + 1, pass_body, 0)


def _pool_first_kernel(B, N):
    upt = B * (HID // CPT) // 32
    tpb = (HID // CPT) // upt
    n_chunks = N // SC_CH
    sub = SC_CH // 16

    @functools.partial(
        pl.kernel,
        out_type=(
            jax.ShapeDtypeStruct((B, HID, N), jnp.float32),
            jax.ShapeDtypeStruct((B, N), jnp.int32),
        ),
        mesh=_sc_mesh(),
        compiler_params=_sc_cp(),
        scratch_types=[
            pltpu.VMEM((CPT * R2,), jnp.float32),
            pltpu.VMEM((SC_CH,), jnp.int32),
            pltpu.VMEM((CPT, SC_CH), jnp.float32),
            pltpu.VMEM((CPT, SC_CH), jnp.float32),
            pltpu.VMEM((N,), jnp.int32),
            pltpu.SemaphoreType.DMA,
            pltpu.SemaphoreType.DMA,
        ],
    )
    def pool(net_hbm, idx_hbm, out_hbm, cnt_hbm,
             acc, idxb, valb, outb, cntb, sem1, sem2):
        wid = lax.axis_index("s") * 2 + lax.axis_index("c")
        b = wid // tpb
        cgbase = (wid % tpb) * (upt * CPT)

        def init(i, _):
            acc[pl.ds(i * 16, 16)] = jnp.full((16,), -jnp.inf, jnp.float32)
            return 0

        def accum(t, first):
            cg = cgbase + t * CPT
            lax.fori_loop(0, CPT * R2 // 16, init, 0)

            def mchunk(mc, _):
                p0 = mc * SC_CH
                d2 = pltpu.async_copy(
                    net_hbm.at[b, pl.ds(cg, CPT), pl.ds(p0, SC_CH)], valb, sem2)
                if first:
                    d1 = pltpu.async_copy(
                        idx_hbm.at[b, 0, pl.ds(p0, SC_CH)], idxb, sem1)
                    d1.wait()
                d2.wait()

                def chunk_first(j, _):
                    ci = idxb[pl.ds(j * 16, 16)]
                    cnt, _last = plsc.scan_count(ci)
                    km = jnp.max(cnt)
                    cntb[pl.ds(p0 + j * 16, 16)] = cnt
                    _max_passes(acc, ci, cnt, km, valb, j)
                    return 0

                def chunk_rest(j, _):
                    ci = cntb_idx(p0, j)
                    cnt = cntb[pl.ds(p0 + j * 16, 16)]
                    km = kmaxb[p0 // 16 + j]
                    _max_passes(acc, ci, cnt, km, valb, j)
                    return 0

                lax.fori_loop(0, sub, chunk_first if first else chunk_rest, 0)
                return 0
            lax.fori_loop(0, n_chunks, mchunk, 0)

        def cntb_idx(p0, j):
            return idxb[pl.ds(j * 16, 16)]

        def gather_back(t, first):
            cg = cgbase + t * CPT

            def gchunk(mc, _):
                p0 = mc * SC_CH
                d1 = pltpu.async_copy(
                    idx_hbm.at[b, 0, pl.ds(p0, SC_CH)], idxb, sem1)
                d1.wait()

                def chunk(j, _):
                    ci = idxb[pl.ds(j * 16, 16)]
                    for c in range(CPT):
                        outb[c, pl.ds(j * 16, 16)] = plsc.load_gather(
                            acc, [ci + (c * R2)])
                    return 0
                lax.fori_loop(0, sub, chunk, 0)
                pltpu.sync_copy(outb,
                                out_hbm.at[b, pl.ds(cg, CPT), pl.ds(p0, SC_CH)])
                return 0
            lax.fori_loop(0, n_chunks, gchunk, 0)

        accum(0, True)
        pltpu.sync_copy(cntb, cnt_hbm.at[b])
        gather_back(0, True)
        for t in range(1, upt):
            cg = cgbase + t * CPT
            lax.fori_loop(0, CPT * R2 // 16, init, 0)

            def mchunk(mc, _, cg=cg):
                p0 = mc * SC_CH
                d1 = pltpu.async_copy(
                    idx_hbm.at[b, 0, pl.ds(p0, SC_CH)], idxb, sem1)
                d2 = pltpu.async_copy(
                    net_hbm.at[b, pl.ds(cg, CPT), pl.ds(p0, SC_CH)], valb, sem2)
                d1.wait()
                d2.wait()

                def chunk(j, _):
                    ci = idxb[pl.ds(j * 16, 16)]
                    cnt = cntb[pl.ds(p0 + j * 16, 16)]
                    km = jnp.max(cnt)
                    _max_passes(acc, ci, cnt, km, valb, j)
                    return 0
                lax.fori_loop(0, sub, chunk, 0)
                return 0
            lax.fori_loop(0, n_chunks, mchunk, 0)
            gather_back(t, False)

    return pool


def _pool_rest_kernel(B, N):
    upt = B * (HID // CPT) // 32
    tpb = (HID // CPT) // upt
    n_chunks = N // SC_CH
    sub = SC_CH // 16

    @functools.partial(
        pl.kernel,
        out_type=jax.ShapeDtypeStruct((B, HID, N), jnp.float32),
        mesh=_sc_mesh(),
        compiler_params=_sc_cp(),
        scratch_types=[
            pltpu.VMEM((CPT * R2,), jnp.float32),
            pltpu.VMEM((SC_CH,), jnp.int32),
            pltpu.VMEM((SC_CH,), jnp.int32),
            pltpu.VMEM((CPT, SC_CH), jnp.float32),
            pltpu.VMEM((CPT, SC_CH), jnp.float32),
            pltpu.SemaphoreType.DMA,
            pltpu.SemaphoreType.DMA,
        ],
    )
    def pool(net_hbm, idx_hbm, cnt_hbm, out_hbm,
             acc, idxb, cntc, valb, outb, sem1, sem2):
        wid = lax.axis_index("s") * 2 + lax.axis_index("c")
        b = wid // tpb
        cgbase = (wid % tpb) * (upt * CPT)

        def init(i, _):
            acc[pl.ds(i * 16, 16)] = jnp.full((16,), -jnp.inf, jnp.float32)
            return 0

        def unit(t, _):
            cg = cgbase + t * CPT
            lax.fori_loop(0, CPT * R2 // 16, init, 0)

            def mchunk(mc, _):
                p0 = mc * SC_CH
                d1 = pltpu.async_copy(
                    idx_hbm.at[b, 0, pl.ds(p0, SC_CH)], idxb, sem1)
                d2 = pltpu.async_copy(
                    cnt_hbm.at[b, pl.ds(p0, SC_CH)], cntc, sem1)
                d4 = pltpu.async_copy(
                    net_hbm.at[b, pl.ds(cg, CPT), pl.ds(p0, SC_CH)], valb, sem2)
                d1.wait()
                d2.wait()
                d4.wait()

                def chunk(j, _):
                    ci = idxb[pl.ds(j * 16, 16)]
                    cnt = cntc[pl.ds(j * 16, 16)]
                    km = jnp.max(cnt)
                    _max_passes(acc, ci, cnt, km, valb, j)
                    return 0
                lax.fori_loop(0, sub, chunk, 0)
                return 0
            lax.fori_loop(0, n_chunks, mchunk, 0)

            def gchunk(mc, _):
                p0 = mc * SC_CH
                d1 = pltpu.async_copy(
                    idx_hbm.at[b, 0, pl.ds(p0, SC_CH)], idxb, sem1)
                d1.wait()

                def chunk(j, _):
                    ci = idxb[pl.ds(j * 16, 16)]
                    for c in range(CPT):
                        outb[c, pl.ds(j * 16, 16)] = plsc.load_gather(
                            acc, [ci + (c * R2)])
                    return 0
                lax.fori_loop(0, sub, chunk, 0)
                pltpu.sync_copy(outb,
                                out_hbm.at[b, pl.ds(cg, CPT), pl.ds(p0, SC_CH)])
                return 0
            lax.fori_loop(0, n_chunks, gchunk, 0)
            return 0

        lax.fori_loop(0, upt, unit, 0)

    return pool


def _mean_kernel(B, N):
    upt = B * (CDIM // CPT) // 32
    tpb = (CDIM // CPT) // upt
    n_chunks = N // SC_CH
    sub = SC_CH // 16
    CNT0 = CPT * R2

    @functools.partial(
        pl.kernel,
        out_type=jax.ShapeDtypeStruct((B, CDIM * R2), jnp.float32),
        mesh=_sc_mesh(),
        compiler_params=_sc_cp(),
        scratch_types=[
            pltpu.VMEM(((CPT + 1) * R2,), jnp.float32),
            pltpu.VMEM((SC_CH,), jnp.int32),
            pltpu.VMEM((SC_CH,), jnp.int32),
            pltpu.VMEM((CPT, SC_CH), jnp.float32),
            pltpu.SemaphoreType.DMA,
            pltpu.SemaphoreType.DMA,
        ],
    )
    def mean(c_hbm, idx_hbm, cnt_hbm, out_hbm,
             acc, idxb, cntc, valb, sem1, sem2):
        wid = lax.axis_index("s") * 2 + lax.axis_index("c")
        b = wid // tpb
        cgbase = (wid % tpb) * (upt * CPT)

        def zero(i, _):
            acc[pl.ds(i * 16, 16)] = jnp.zeros((16,), jnp.float32)
            return 0

        def unit(t, first):
            cg = cgbase + t * CPT
            lax.fori_loop(0, ((CPT + 1) if first else CPT) * R2 // 16, zero, 0)

            def mchunk(mc, _):
                p0 = mc * SC_CH
                d1 = pltpu.async_copy(
                    idx_hbm.at[b, 0, pl.ds(p0, SC_CH)], idxb, sem1)
                d2 = pltpu.async_copy(
                    cnt_hbm.at[b, pl.ds(p0, SC_CH)], cntc, sem1)
                d4 = pltpu.async_copy(
                    c_hbm.at[b, pl.ds(cg, CPT), pl.ds(p0, SC_CH)], valb, sem2)
                d1.wait()
                d2.wait()
                d4.wait()

                def chunk(j, _):
                    ci = idxb[pl.ds(j * 16, 16)]
                    cnt = cntc[pl.ds(j * 16, 16)]
                    km = jnp.max(cnt)

                    def pass_body(k, _):
                        sel = cnt == k
                        for c in range(CPT):
                            v = valb[c, pl.ds(j * 16, 16)]
                            cic = ci + (c * R2)
                            cur = plsc.load_gather(acc, [cic], mask=sel)
                            plsc.store_scatter(acc, [cic], cur + v, mask=sel)
                        if first:
                            cic = ci + CNT0
                            cur = plsc.load_gather(acc, [cic], mask=sel)
                            plsc.store_scatter(
                                acc, [cic], cur + jnp.ones((16,), jnp.float32),
                                mask=sel)
                        return 0
                    lax.fori_loop(1, km + 1, pass_body, 0)
                    return 0
                lax.fori_loop(0, sub, chunk, 0)
                return 0
            lax.fori_loop(0, n_chunks, mchunk, 0)

            if first:
                def recip(i, _):
                    n = acc[pl.ds(CNT0 + i * 16, 16)]
                    acc[pl.ds(CNT0 + i * 16, 16)] = 1.0 / jnp.maximum(n, 1.0)
                    return 0
                lax.fori_loop(0, R2 // 16, recip, 0)

            def fin(i, _):
                inv = acc[pl.ds(CNT0 + i * 16, 16)]
                for c in range(CPT):
                    s = acc[pl.ds(c * R2 + i * 16, 16)]
                    acc[pl.ds(c * R2 + i * 16, 16)] = s * inv
                return 0
            lax.fori_loop(0, R2 // 16, fin, 0)

            pltpu.sync_copy(acc.at[pl.ds(0, CPT * R2)],
                            out_hbm.at[b, pl.ds(cg * R2, CPT * R2)])

        unit(0, True)
        for t in range(1, upt):
            unit(t, False)

    return mean


def kernel(inputs, fc_pos_W, fc_pos_b, blocks_W0, blocks_b0, blocks_W1,
           blocks_b1, blocks_Ws, fc_c_W, fc_c_b):
    B, N, _ = inputs.shape

    b_pos = fc_pos_b.reshape(-1, 1)
    b0 = blocks_b0.reshape(NBLK, -1, 1)
    b1 = blocks_b1.reshape(NBLK, -1, 1)
    b_c = fc_c_b.reshape(-1, 1)

    netT, idx = _tc_s0(inputs, fc_pos_W, b_pos,
                       blocks_W0[0], b0[0], blocks_W1[0], b1[0], blocks_Ws[0])

    poolT, cnt = _pool_first_kernel(B, N)(netT, idx)
    netT = _tc_block(netT, poolT, blocks_W0[1], b0[1], blocks_W1[1],
                     b1[1], blocks_Ws[1])

    pool_rest = _pool_rest_kernel(B, N)
    for i in range(2, NBLK):
        poolT = pool_rest(netT, idx, cnt)
        netT = _tc_block(netT, poolT, blocks_W0[i], b0[i], blocks_W1[i],
                         b1[i], blocks_Ws[i])

    cT = _tc_s5(netT, fc_c_W, b_c)
    plane = _mean_kernel(B, N)(cT, idx, cnt)
    return plane.reshape(B, CDIM, RESO, RESO)


# speedup vs baseline: 9.7911x; 1.2344x over previous
import functools

import jax
import jax.numpy as jnp
from jax import lax
from jax.experimental import pallas as pl
from jax.experimental.pallas import tpu as pltpu
from jax.experimental.pallas import tpu_sc as plsc

RESO = 128
R2 = RESO * RESO
HID = 32
CDIM = 32
NBLK = 5

CHT = 2048
SC_CH = 2048
CPT = 4


def _s0_body(x_ref, W_ref, b_ref, W0_ref, b0_ref, W1_ref, b1_ref, Ws_ref,
             net_ref, idx_ref):
    x = x_ref[0]
    xi = (x[:, 0] * float(RESO)).astype(jnp.int32)
    yi = (x[:, 1] * float(RESO)).astype(jnp.int32)
    idx_ref[0, 0] = xi + RESO * yi

    W = W_ref[...]
    net = lax.dot_general(W, x, (((1,), (1,)), ((), ())),
                          preferred_element_type=jnp.float32)
    net = net + b_ref[...]
    xr = jnp.maximum(net, 0.0)
    h = jnp.dot(W0_ref[...], xr, preferred_element_type=jnp.float32) + b0_ref[...]
    d = jnp.dot(W1_ref[...], jnp.maximum(h, 0.0),
                preferred_element_type=jnp.float32) + b1_ref[...]
    net_ref[0] = jnp.dot(Ws_ref[...], net, preferred_element_type=jnp.float32) + d


def _blk_body(net_ref, pool_ref, W0_ref, b0_ref, W1_ref, b1_ref, Ws_ref, out_ref):
    x = jnp.concatenate([net_ref[0], pool_ref[0]], axis=0)
    xr = jnp.maximum(x, 0.0)
    h = jnp.dot(W0_ref[...], xr, preferred_element_type=jnp.float32) + b0_ref[...]
    d = jnp.dot(W1_ref[...], jnp.maximum(h, 0.0),
                preferred_element_type=jnp.float32) + b1_ref[...]
    out_ref[0] = jnp.dot(Ws_ref[...], x, preferred_element_type=jnp.float32) + d


def _s5_body(net_ref, W_ref, b_ref, out_ref):
    xr = jnp.maximum(net_ref[0], 0.0)
    out_ref[0] = jnp.dot(W_ref[...], xr, preferred_element_type=jnp.float32) + b_ref[...]


def _full(shape):
    return pl.BlockSpec(shape, lambda b, i: (0,) * len(shape))


def _tc_s0(inputs, fc_pos_W, fc_pos_b, W0, b0, W1, b1, Ws, boff, Bh):
    _, N, _ = inputs.shape
    B = Bh
    grid = (B, N // CHT)
    return pl.pallas_call(
        _s0_body,
        grid=grid,
        in_specs=[
            pl.BlockSpec((1, CHT, 3), lambda b, i, boff=boff: (b + boff, i, 0)),
            _full(fc_pos_W.shape), _full(fc_pos_b.shape),
            _full(W0.shape), _full(b0.shape),
            _full(W1.shape), _full(b1.shape),
            _full(Ws.shape),
        ],
        out_specs=[
            pl.BlockSpec((1, HID, CHT), lambda b, i: (b, 0, i)),
            pl.BlockSpec((1, 1, CHT), lambda b, i: (b, 0, i)),
        ],
        out_shape=[
            jax.ShapeDtypeStruct((B, HID, N), jnp.float32),
            jax.ShapeDtypeStruct((B, 1, N), jnp.int32),
        ],
    )(inputs, fc_pos_W, fc_pos_b, W0, b0, W1, b1, Ws)


def _tc_block(netT, poolT, W0, b0, W1, b1, Ws):
    B, _, N = netT.shape
    grid = (B, N // CHT)
    return pl.pallas_call(
        _blk_body,
        grid=grid,
        in_specs=[
            pl.BlockSpec((1, HID, CHT), lambda b, i: (b, 0, i)),
            pl.BlockSpec((1, HID, CHT), lambda b, i: (b, 0, i)),
            _full(W0.shape), _full(b0.shape),
            _full(W1.shape), _full(b1.shape),
            _full(Ws.shape),
        ],
        out_specs=pl.BlockSpec((1, HID, CHT), lambda b, i: (b, 0, i)),
        out_shape=jax.ShapeDtypeStruct((B, HID, N), jnp.float32),
    )(netT, poolT, W0, b0, W1, b1, Ws)


def _tc_s5(netT, W, b):
    B, _, N = netT.shape
    grid = (B, N // CHT)
    return pl.pallas_call(
        _s5_body,
        grid=grid,
        in_specs=[
            pl.BlockSpec((1, HID, CHT), lambda b, i: (b, 0, i)),
            _full(W.shape), _full(b.shape),
        ],
        out_specs=pl.BlockSpec((1, CDIM, CHT), lambda b, i: (b, 0, i)),
        out_shape=jax.ShapeDtypeStruct((B, CDIM, N), jnp.float32),
    )(netT, W, b)


def _sc_mesh():
    return plsc.VectorSubcoreMesh(core_axis_name="c", subcore_axis_name="s")


def _sc_cp():
    return pltpu.CompilerParams(needs_layout_passes=False)


def _max_passes(acc, ci, cnt, km, valb, j):
    def pass_body(k, _):
        sel = cnt == k
        for c in range(CPT):
            v = valb[c, pl.ds(j * 16, 16)]
            cic = ci + (c * R2)
            cur = plsc.load_gather(acc, [cic], mask=sel)
            plsc.store_scatter(acc, [cic], jnp.maximum(cur, v), mask=sel)
        return 0
    lax.fori_loop(1, km + 1, pass_body, 0)


def _pool_first_kernel(B, N):
    upt = B * (HID // CPT) // 32
    tpb = (HID // CPT) // upt
    n_chunks = N // SC_CH
    sub = SC_CH // 16

    @functools.partial(
        pl.kernel,
        out_type=(
            jax.ShapeDtypeStruct((B, HID, N), jnp.float32),
            jax.ShapeDtypeStruct((B, N), jnp.int32),
        ),
        mesh=_sc_mesh(),
        compiler_params=_sc_cp(),
        scratch_types=[
            pltpu.VMEM((CPT * R2,), jnp.float32),
            pltpu.VMEM((SC_CH,), jnp.int32),
            pltpu.VMEM((CPT, SC_CH), jnp.float32),
            pltpu.VMEM((CPT, SC_CH), jnp.float32),
            pltpu.VMEM((N,), jnp.int32),
            pltpu.SemaphoreType.DMA,
            pltpu.SemaphoreType.DMA,
        ],
    )
    def pool(net_hbm, idx_hbm, out_hbm, cnt_hbm,
             acc, idxb, valb, outb, cntb, sem1, sem2):
        wid = lax.axis_index("s") * 2 + lax.axis_index("c")
        b = wid // tpb
        cgbase = (wid % tpb) * (upt * CPT)

        def init(i, _):
            acc[pl.ds(i * 16, 16)] = jnp.full((16,), -jnp.inf, jnp.float32)
            return 0

        def accum(t, first):
            cg = cgbase + t * CPT
            lax.fori_loop(0, CPT * R2 // 16, init, 0)

            def mchunk(mc, _):
                p0 = mc * SC_CH
                d2 = pltpu.async_copy(
                    net_hbm.at[b, pl.ds(cg, CPT), pl.ds(p0, SC_CH)], valb, sem2)
                if first:
                    d1 = pltpu.async_copy(
                        idx_hbm.at[b, 0, pl.ds(p0, SC_CH)], idxb, sem1)
                    d1.wait()
                d2.wait()

                def chunk_first(j, _):
                    ci = idxb[pl.ds(j * 16, 16)]
                    cnt, _last = plsc.scan_count(ci)
                    km = jnp.max(cnt)
                    cntb[pl.ds(p0 + j * 16, 16)] = cnt
                    _max_passes(acc, ci, cnt, km, valb, j)
                    return 0

                def chunk_rest(j, _):
                    ci = cntb_idx(p0, j)
                    cnt = cntb[pl.ds(p0 + j * 16, 16)]
                    km = kmaxb[p0 // 16 + j]
                    _max_passes(acc, ci, cnt, km, valb, j)
                    return 0

                lax.fori_loop(0, sub, chunk_first if first else chunk_rest, 0)
                return 0
            lax.fori_loop(0, n_chunks, mchunk, 0)

        def cntb_idx(p0, j):
            return idxb[pl.ds(j * 16, 16)]

        def gather_back(t, first):
            cg = cgbase + t * CPT

            def gchunk(mc, _):
                p0 = mc * SC_CH
                d1 = pltpu.async_copy(
                    idx_hbm.at[b, 0, pl.ds(p0, SC_CH)], idxb, sem1)
                d1.wait()

                def chunk(j, _):
                    ci = idxb[pl.ds(j * 16, 16)]
                    for c in range(CPT):
                        outb[c, pl.ds(j * 16, 16)] = plsc.load_gather(
                            acc, [ci + (c * R2)])
                    return 0
                lax.fori_loop(0, sub, chunk, 0)
                pltpu.sync_copy(outb,
                                out_hbm.at[b, pl.ds(cg, CPT), pl.ds(p0, SC_CH)])
                return 0
            lax.fori_loop(0, n_chunks, gchunk, 0)

        accum(0, True)
        pltpu.sync_copy(cntb, cnt_hbm.at[b])
        gather_back(0, True)
        for t in range(1, upt):
            cg = cgbase + t * CPT
            lax.fori_loop(0, CPT * R2 // 16, init, 0)

            def mchunk(mc, _, cg=cg):
                p0 = mc * SC_CH
                d1 = pltpu.async_copy(
                    idx_hbm.at[b, 0, pl.ds(p0, SC_CH)], idxb, sem1)
                d2 = pltpu.async_copy(
                    net_hbm.at[b, pl.ds(cg, CPT), pl.ds(p0, SC_CH)], valb, sem2)
                d1.wait()
                d2.wait()

                def chunk(j, _):
                    ci = idxb[pl.ds(j * 16, 16)]
                    cnt = cntb[pl.ds(p0 + j * 16, 16)]
                    km = jnp.max(cnt)
                    _max_passes(acc, ci, cnt, km, valb, j)
                    return 0
                lax.fori_loop(0, sub, chunk, 0)
                return 0
            lax.fori_loop(0, n_chunks, mchunk, 0)
            gather_back(t, False)

    return pool


def _pool_rest_kernel(B, N):
    upt = B * (HID // CPT) // 32
    tpb = (HID // CPT) // upt
    n_chunks = N // SC_CH
    sub = SC_CH // 16

    @functools.partial(
        pl.kernel,
        out_type=jax.ShapeDtypeStruct((B, HID, N), jnp.float32),
        mesh=_sc_mesh(),
        compiler_params=_sc_cp(),
        scratch_types=[
            pltpu.VMEM((CPT * R2,), jnp.float32),
            pltpu.VMEM((SC_CH,), jnp.int32),
            pltpu.VMEM((SC_CH,), jnp.int32),
            pltpu.VMEM((CPT, SC_CH), jnp.float32),
            pltpu.VMEM((CPT, SC_CH), jnp.float32),
            pltpu.SemaphoreType.DMA,
            pltpu.SemaphoreType.DMA,
        ],
    )
    def pool(net_hbm, idx_hbm, cnt_hbm, out_hbm,
             acc, idxb, cntc, valb, outb, sem1, sem2):
        wid = lax.axis_index("s") * 2 + lax.axis_index("c")
        b = wid // tpb
        cgbase = (wid % tpb) * (upt * CPT)

        def init(i, _):
            acc[pl.ds(i * 16, 16)] = jnp.full((16,), -jnp.inf, jnp.float32)
            return 0

        def unit(t, _):
            cg = cgbase + t * CPT
            lax.fori_loop(0, CPT * R2 // 16, init, 0)

            def mchunk(mc, _):
                p0 = mc * SC_CH
                d1 = pltpu.async_copy(
                    idx_hbm.at[b, 0, pl.ds(p0, SC_CH)], idxb, sem1)
                d2 = pltpu.async_copy(
                    cnt_hbm.at[b, pl.ds(p0, SC_CH)], cntc, sem1)
                d4 = pltpu.async_copy(
                    net_hbm.at[b, pl.ds(cg, CPT), pl.ds(p0, SC_CH)], valb, sem2)
                d1.wait()
                d2.wait()
                d4.wait()

                def chunk(j, _):
                    ci = idxb[pl.ds(j * 16, 16)]
                    cnt = cntc[pl.ds(j * 16, 16)]
                    km = jnp.max(cnt)
                    _max_passes(acc, ci, cnt, km, valb, j)
                    return 0
                lax.fori_loop(0, sub, chunk, 0)
                return 0
            lax.fori_loop(0, n_chunks, mchunk, 0)

            def gchunk(mc, _):
                p0 = mc * SC_CH
                d1 = pltpu.async_copy(
                    idx_hbm.at[b, 0, pl.ds(p0, SC_CH)], idxb, sem1)
                d1.wait()

                def chunk(j, _):
                    ci = idxb[pl.ds(j * 16, 16)]
                    for c in range(CPT):
                        outb[c, pl.ds(j * 16, 16)] = plsc.load_gather(
                            acc, [ci + (c * R2)])
                    return 0
                lax.fori_loop(0, sub, chunk, 0)
                pltpu.sync_copy(outb,
                                out_hbm.at[b, pl.ds(cg, CPT), pl.ds(p0, SC_CH)])
                return 0
            lax.fori_loop(0, n_chunks, gchunk, 0)
            return 0

        lax.fori_loop(0, upt, unit, 0)

    return pool


def _mean_kernel(B, N):
    upt = B * (CDIM // CPT) // 32
    tpb = (CDIM // CPT) // upt
    n_chunks = N // SC_CH
    sub = SC_CH // 16
    CNT0 = CPT * R2

    @functools.partial(
        pl.kernel,
        out_type=jax.ShapeDtypeStruct((B, CDIM * R2), jnp.float32),
        mesh=_sc_mesh(),
        compiler_params=_sc_cp(),
        scratch_types=[
            pltpu.VMEM(((CPT + 1) * R2,), jnp.float32),
            pltpu.VMEM((SC_CH,), jnp.int32),
            pltpu.VMEM((SC_CH,), jnp.int32),
            pltpu.VMEM((CPT, SC_CH), jnp.float32),
            pltpu.SemaphoreType.DMA,
            pltpu.SemaphoreType.DMA,
        ],
    )
    def mean(c_hbm, idx_hbm, cnt_hbm, out_hbm,
             acc, idxb, cntc, valb, sem1, sem2):
        wid = lax.axis_index("s") * 2 + lax.axis_index("c")
        b = wid // tpb
        cgbase = (wid % tpb) * (upt * CPT)

        def zero(i, _):
            acc[pl.ds(i * 16, 16)] = jnp.zeros((16,), jnp.float32)
            return 0

        def unit(t, first):
            cg = cgbase + t * CPT
            lax.fori_loop(0, ((CPT + 1) if first else CPT) * R2 // 16, zero, 0)

            def mchunk(mc, _):
                p0 = mc * SC_CH
                d1 = pltpu.async_copy(
                    idx_hbm.at[b, 0, pl.ds(p0, SC_CH)], idxb, sem1)
                d2 = pltpu.async_copy(
                    cnt_hbm.at[b, pl.ds(p0, SC_CH)], cntc, sem1)
                d4 = pltpu.async_copy(
                    c_hbm.at[b, pl.ds(cg, CPT), pl.ds(p0, SC_CH)], valb, sem2)
                d1.wait()
                d2.wait()
                d4.wait()

                def chunk(j, _):
                    ci = idxb[pl.ds(j * 16, 16)]
                    cnt = cntc[pl.ds(j * 16, 16)]
                    km = jnp.max(cnt)

                    def pass_body(k, _):
                        sel = cnt == k
                        for c in range(CPT):
                            v = valb[c, pl.ds(j * 16, 16)]
                            cic = ci + (c * R2)
                            cur = plsc.load_gather(acc, [cic], mask=sel)
                            plsc.store_scatter(acc, [cic], cur + v, mask=sel)
                        if first:
                            cic = ci + CNT0
                            cur = plsc.load_gather(acc, [cic], mask=sel)
                            plsc.store_scatter(
                                acc, [cic], cur + jnp.ones((16,), jnp.float32),
                                mask=sel)
                        return 0
                    lax.fori_loop(1, km + 1, pass_body, 0)
                    return 0
                lax.fori_loop(0, sub, chunk, 0)
                return 0
            lax.fori_loop(0, n_chunks, mchunk, 0)

            if first:
                def recip(i, _):
                    n = acc[pl.ds(CNT0 + i * 16, 16)]
                    acc[pl.ds(CNT0 + i * 16, 16)] = 1.0 / jnp.maximum(n, 1.0)
                    return 0
                lax.fori_loop(0, R2 // 16, recip, 0)

            def fin(i, _):
                inv = acc[pl.ds(CNT0 + i * 16, 16)]
                for c in range(CPT):
                    s = acc[pl.ds(c * R2 + i * 16, 16)]
                    acc[pl.ds(c * R2 + i * 16, 16)] = s * inv
                return 0
            lax.fori_loop(0, R2 // 16, fin, 0)

            pltpu.sync_copy(acc.at[pl.ds(0, CPT * R2)],
                            out_hbm.at[b, pl.ds(cg * R2, CPT * R2)])

        unit(0, True)
        for t in range(1, upt):
            unit(t, False)

    return mean


def kernel(inputs, fc_pos_W, fc_pos_b, blocks_W0, blocks_b0, blocks_W1,
           blocks_b1, blocks_Ws, fc_c_W, fc_c_b):
    B, N, _ = inputs.shape

    b_pos = fc_pos_b.reshape(-1, 1)
    b0 = blocks_b0.reshape(NBLK, -1, 1)
    b1 = blocks_b1.reshape(NBLK, -1, 1)
    b_c = fc_c_b.reshape(-1, 1)

    Bh = B // 2
    pool_first = _pool_first_kernel(Bh, N)
    pool_rest = _pool_rest_kernel(Bh, N)
    mean_k = _mean_kernel(Bh, N)

    planes = []
    for h in range(2):
        netT, idx = _tc_s0(inputs, fc_pos_W, b_pos,
                           blocks_W0[0], b0[0], blocks_W1[0], b1[0],
                           blocks_Ws[0], h * Bh, Bh)

        poolT, cnt = pool_first(netT, idx)
        netT = _tc_block(netT, poolT, blocks_W0[1], b0[1], blocks_W1[1],
                         b1[1], blocks_Ws[1])

        for i in range(2, NBLK):
            poolT = pool_rest(netT, idx, cnt)
            netT = _tc_block(netT, poolT, blocks_W0[i], b0[i], blocks_W1[i],
                             b1[i], blocks_Ws[i])

        cT = _tc_s5(netT, fc_c_W, b_c)
        planes.append(mean_k(cT, idx, cnt))

    plane = jnp.concatenate(planes, axis=0)
    return plane.reshape(B, CDIM, RESO, RESO)
